# baseline (speedup 1.0000x reference)
import jax
import jax.numpy as jnp
from jax import lax
from jax.experimental import pallas as pl
from jax.experimental.pallas import tpu as pltpu
from jax.experimental.pallas import tpu_sc as plsc
import functools

NUM_EMB = 1_000_000
DIM = 64
BATCH = 16384
FIELDS = 26
TOT = BATCH * FIELDS

NC = 2
NS = 16
NW = NC * NS
BPW = TOT // NW
CHUNK = 512
NCHUNK = BPW // CHUNK
NGROUP = NCHUNK // 2

assert BPW * NW == TOT and CHUNK * NCHUNK == BPW and NGROUP * 2 == NCHUNK

_mesh = plsc.VectorSubcoreMesh(
    core_axis_name="c", subcore_axis_name="s", num_cores=NC, num_subcores=NS
)


@functools.partial(
    pl.kernel,
    out_type=jax.ShapeDtypeStruct((TOT, DIM), jnp.float32),
    mesh=_mesh,
    scratch_types=[
        pltpu.VMEM((BPW,), jnp.int32),
        pltpu.VMEM((CHUNK, DIM), jnp.float32),
        pltpu.VMEM((CHUNK, DIM), jnp.float32),
        pltpu.SemaphoreType.DMA,
        pltpu.SemaphoreType.DMA,
    ],
    compiler_params=pltpu.CompilerParams(use_tc_tiling_on_sc=False),
)
def _sc_gather(tbl, idx, out, idx_v, rows0, rows1, sem0, sem1):
    wid = lax.axis_index("s") * NC + lax.axis_index("c")
    base = wid * BPW
    pltpu.sync_copy(idx.at[pl.ds(base, BPW)], idx_v)

    bufs = (rows0, rows1)
    sems = (sem0, sem1)

    def start(c, b):
        pltpu.make_async_copy(
            tbl.at[idx_v.at[pl.ds(c * CHUNK, CHUNK)]], bufs[b], sems[b]
        ).start()

    def finish(c, b):
        pltpu.make_async_copy(
            tbl.at[idx_v.at[pl.ds(c * CHUNK, CHUNK)]], bufs[b], sems[b]
        ).wait()
        pltpu.sync_copy(bufs[b], out.at[pl.ds(base + c * CHUNK, CHUNK)])

    start(0, 0)
    start(1, 1)

    @pl.loop(0, NGROUP - 1)
    def _(g):
        for b in range(2):
            c = 2 * g + b
            finish(c, b)
            start(c + 2, b)

    for b in range(2):
        finish((NGROUP - 1) * 2 + b, b)


def kernel(input, weight):
    idx = input.reshape(-1).astype(jnp.int32)
    out = _sc_gather(weight, idx)
    return out.reshape(BATCH, FIELDS, DIM)
